# baseline (speedup 1.0000x reference)
import functools

import jax
import jax.numpy as jnp
from jax import lax
from jax.experimental import pallas as pl
from jax.experimental.pallas import tpu as pltpu
from jax.experimental.pallas import tpu_sc as plsc

N = 10000
E = 160000
D = 256
DH = 128
NC = 2
NS = 16
L = 16
CK = 128
NCHUNK = 79
EPAD = NS * NCHUNK * CK
RPT = 624
TAIL = N - NS * RPT
DEG_PAD = 10240
DEG_SLICE = DEG_PAD // NS

_mesh = plsc.VectorSubcoreMesh(
    core_axis_name="c", subcore_axis_name="s", num_cores=NC, num_subcores=NS)


def _deg_body(col_hbm, ew_hbm, out_hbm, col_v, ew_v, zbuf, acc):
    c = lax.axis_index("c")
    s = lax.axis_index("s")

    def zero(i, carry):
        zbuf[pl.ds(i * L, L)] = jnp.zeros((L,), jnp.float32)
        return carry

    lax.fori_loop(0, DEG_SLICE // L, zero, 0)
    pltpu.sync_copy(zbuf, acc.at[pl.ds(s * DEG_SLICE, DEG_SLICE)])
    pltpu.sync_copy(col_hbm.at[s], col_v)
    pltpu.sync_copy(ew_hbm.at[s], ew_v)
    plsc.subcore_barrier()

    half = (NCHUNK + 1) // 2

    def go(i, carry):
        ci = c * half + i

        @pl.when(ci < NCHUNK)
        def _():
            pltpu.sync_copy(ew_v.at[ci], acc.at[col_v.at[ci]], add=True)

        return carry

    lax.fori_loop(0, half, go, 0)
    plsc.subcore_barrier()
    pltpu.sync_copy(acc.at[pl.ds(s * DEG_SLICE, DEG_SLICE)],
                    out_hbm.at[pl.ds(c * DEG_PAD + s * DEG_SLICE, DEG_SLICE)])


_deg_kernel = pl.kernel(
    _deg_body,
    out_type=jax.ShapeDtypeStruct((NC * DEG_PAD,), jnp.float32),
    mesh=_mesh,
    scratch_types=[
        pltpu.VMEM((NCHUNK, CK), jnp.int32),
        pltpu.VMEM((NCHUNK, CK), jnp.float32),
        pltpu.VMEM((DEG_SLICE,), jnp.float32),
        pltpu.VMEM_SHARED((DEG_PAD,), jnp.float32),
    ],
)


def _layer_body(y_hbm, row_hbm, col_hbm, ew_hbm, out_hbm,
                row_v, col_v, ew_v, msg_v, gsem, acc):
    c = lax.axis_index("c")
    s = lax.axis_index("s")

    pltpu.sync_copy(y_hbm.at[pl.ds(c * N + s * RPT, RPT)],
                    acc.at[pl.ds(s * RPT, RPT)])

    @pl.when(s == 0)
    def _():
        pltpu.sync_copy(y_hbm.at[pl.ds(c * N + NS * RPT, TAIL)],
                        acc.at[pl.ds(NS * RPT, TAIL)])

    pltpu.sync_copy(row_hbm.at[s], row_v)
    pltpu.sync_copy(col_hbm.at[s], col_v)
    pltpu.sync_copy(ew_hbm.at[s], ew_v)

    offv = jnp.full((L,), c * N, jnp.int32)

    def addoff(i, carry):
        for u in range(CK // L):
            sl = (i, pl.ds(u * L, L))
            row_v[sl] = row_v[sl] + offv
        return carry

    lax.fori_loop(0, NCHUNK, addoff, 0)
    plsc.subcore_barrier()

    def chunk(i, carry):
        for g in range(2):
            pltpu.async_copy(y_hbm.at[row_v.at[i, pl.ds(g * 64, 64)]],
                             msg_v.at[pl.ds(g * 64, 64)], gsem)
        for g in range(2):
            pltpu.make_async_copy(y_hbm.at[row_v.at[i, pl.ds(g * 64, 64)]],
                                  msg_v.at[pl.ds(g * 64, 64)], gsem).wait()

        def scale(g, carry2):
            ew16 = ew_v[i, pl.ds(g * L, L)]
            for lane in range(L):
                wv = jnp.full((L,), ew16[lane], jnp.float32)
                j = g * L + lane
                for u in range(DH // L):
                    sl = (j, pl.ds(u * L, L))
                    msg_v[sl] = msg_v[sl] * wv
            return carry2

        lax.fori_loop(0, CK // L, scale, 0)
        pltpu.sync_copy(msg_v, acc.at[col_v.at[i]], add=True)
        return carry

    lax.fori_loop(0, NCHUNK, chunk, 0)
    plsc.subcore_barrier()
    pltpu.sync_copy(acc.at[pl.ds(s * RPT, RPT)],
                    out_hbm.at[pl.ds(c * N + s * RPT, RPT)])

    @pl.when(s == 0)
    def _():
        pltpu.sync_copy(acc.at[pl.ds(NS * RPT, TAIL)],
                        out_hbm.at[pl.ds(c * N + NS * RPT, TAIL)])


_layer_kernel = pl.kernel(
    _layer_body,
    out_type=jax.ShapeDtypeStruct((NC * N, DH), jnp.float32),
    mesh=_mesh,
    scratch_types=[
        pltpu.VMEM((NCHUNK, CK), jnp.int32),
        pltpu.VMEM((NCHUNK, CK), jnp.int32),
        pltpu.VMEM((NCHUNK, CK), jnp.float32),
        pltpu.VMEM((CK, DH), jnp.float32),
        pltpu.SemaphoreType.DMA,
        pltpu.VMEM_SHARED((N, DH), jnp.float32),
    ],
)


BR = 1000


def _first_body(x_ref, w_ref, dinv_ref, out_ref):
    xw = jnp.dot(x_ref[...], w_ref[...], preferred_element_type=jnp.float32)
    y = xw * dinv_ref[...]
    out_ref[0] = y[:, :DH]
    out_ref[1] = y[:, DH:]


_first_kernel = pl.pallas_call(
    _first_body,
    grid=(N // BR,),
    in_specs=[
        pl.BlockSpec((BR, D), lambda i: (i, 0)),
        pl.BlockSpec((D, D), lambda i: (0, 0)),
        pl.BlockSpec((BR, 1), lambda i: (i, 0)),
    ],
    out_specs=pl.BlockSpec((NC, BR, DH), lambda i: (0, i, 0)),
    out_shape=jax.ShapeDtypeStruct((NC, N, DH), jnp.float32),
)


def _mid_body(a_ref, dinv_ref, b_ref, w_ref, out_ref):
    d = dinv_ref[...]
    h0 = jnp.maximum(a_ref[0] * d + b_ref[:, :DH], 0.0)
    h1 = jnp.maximum(a_ref[1] * d + b_ref[:, DH:], 0.0)
    y = (jnp.dot(h0, w_ref[:DH, :], preferred_element_type=jnp.float32)
         + jnp.dot(h1, w_ref[DH:, :], preferred_element_type=jnp.float32)) * d
    out_ref[0] = y[:, :DH]
    out_ref[1] = y[:, DH:]


_mid_kernel = pl.pallas_call(
    _mid_body,
    grid=(N // BR,),
    in_specs=[
        pl.BlockSpec((NC, BR, DH), lambda i: (0, i, 0)),
        pl.BlockSpec((BR, 1), lambda i: (i, 0)),
        pl.BlockSpec((1, D), lambda i: (0, 0)),
        pl.BlockSpec((D, D), lambda i: (0, 0)),
    ],
    out_specs=pl.BlockSpec((NC, BR, DH), lambda i: (0, i, 0)),
    out_shape=jax.ShapeDtypeStruct((NC, N, DH), jnp.float32),
)


def _last_body(a_ref, dinv_ref, b_ref, out_ref):
    d = dinv_ref[...]
    out_ref[:, :DH] = jax.nn.sigmoid(a_ref[0] * d + b_ref[:, :DH])
    out_ref[:, DH:] = jax.nn.sigmoid(a_ref[1] * d + b_ref[:, DH:])


_last_kernel = pl.pallas_call(
    _last_body,
    grid=(N // BR,),
    in_specs=[
        pl.BlockSpec((NC, BR, DH), lambda i: (0, i, 0)),
        pl.BlockSpec((BR, 1), lambda i: (i, 0)),
        pl.BlockSpec((1, D), lambda i: (0, 0)),
    ],
    out_specs=pl.BlockSpec((BR, D), lambda i: (i, 0)),
    out_shape=jax.ShapeDtypeStruct((N, D), jnp.float32),
)


def kernel(x, edge_index, edge_attr, W1, b1, W2, b2, W3, b3):
    row = edge_index[0].astype(jnp.int32)
    col = edge_index[1].astype(jnp.int32)
    ew = edge_attr.astype(jnp.float32)
    pad = EPAD - E
    rowp = jnp.concatenate([row, jnp.zeros((pad,), jnp.int32)]
                           ).reshape(NS, NCHUNK, CK)
    colp = jnp.concatenate([col, jnp.zeros((pad,), jnp.int32)]
                           ).reshape(NS, NCHUNK, CK)
    ewp = jnp.concatenate([ew, jnp.zeros((pad,), jnp.float32)]
                          ).reshape(NS, NCHUNK, CK)

    degp = _deg_kernel(colp, ewp).reshape(NC, DEG_PAD)
    deg = degp[0, :N] + degp[1, :N] + 1.0
    dinv = jnp.where(deg > 0, lax.rsqrt(jnp.maximum(deg, 1e-30)),
                     0.0).reshape(N, 1)

    y1 = _first_kernel(x, W1, dinv).reshape(NC * N, DH)
    a1 = _layer_kernel(y1, rowp, colp, ewp).reshape(NC, N, DH)
    y2 = _mid_kernel(a1, dinv, b1.reshape(1, D), W2).reshape(NC * N, DH)
    a2 = _layer_kernel(y2, rowp, colp, ewp).reshape(NC, N, DH)
    y3 = _mid_kernel(a2, dinv, b2.reshape(1, D), W3).reshape(NC * N, DH)
    a3 = _layer_kernel(y3, rowp, colp, ewp).reshape(NC, N, DH)
    return _last_kernel(a3, dinv, b3.reshape(1, D))


# speedup vs baseline: 1.1961x; 1.1961x over previous
import functools

import jax
import jax.numpy as jnp
from jax import lax
from jax.experimental import pallas as pl
from jax.experimental.pallas import tpu as pltpu
from jax.experimental.pallas import tpu_sc as plsc

N = 10000
E = 160000
D = 256
DH = 128
NC = 2
NS = 16
L = 16
CK = 128
NCHUNK = 79
EPAD = NS * NCHUNK * CK
RPT = 624
TAIL = N - NS * RPT
DEG_PAD = 10240
DEG_SLICE = DEG_PAD // NS

_mesh = plsc.VectorSubcoreMesh(
    core_axis_name="c", subcore_axis_name="s", num_cores=NC, num_subcores=NS)


def _deg_body(col_hbm, ew_hbm, out_hbm, col_v, ew_v, zbuf, acc):
    c = lax.axis_index("c")
    s = lax.axis_index("s")

    def zero(i, carry):
        zbuf[pl.ds(i * L, L)] = jnp.zeros((L,), jnp.float32)
        return carry

    lax.fori_loop(0, DEG_SLICE // L, zero, 0)
    pltpu.sync_copy(zbuf, acc.at[pl.ds(s * DEG_SLICE, DEG_SLICE)])
    pltpu.sync_copy(col_hbm.at[s], col_v)
    pltpu.sync_copy(ew_hbm.at[s], ew_v)
    plsc.subcore_barrier()

    half = (NCHUNK + 1) // 2

    def go(i, carry):
        ci = c * half + i

        @pl.when(ci < NCHUNK)
        def _():
            pltpu.sync_copy(ew_v.at[ci], acc.at[col_v.at[ci]], add=True)

        return carry

    lax.fori_loop(0, half, go, 0)
    plsc.subcore_barrier()
    pltpu.sync_copy(acc.at[pl.ds(s * DEG_SLICE, DEG_SLICE)],
                    out_hbm.at[pl.ds(c * DEG_PAD + s * DEG_SLICE, DEG_SLICE)])


_deg_kernel = pl.kernel(
    _deg_body,
    out_type=jax.ShapeDtypeStruct((NC * DEG_PAD,), jnp.float32),
    mesh=_mesh,
    scratch_types=[
        pltpu.VMEM((NCHUNK, CK), jnp.int32),
        pltpu.VMEM((NCHUNK, CK), jnp.float32),
        pltpu.VMEM((DEG_SLICE,), jnp.float32),
        pltpu.VMEM_SHARED((DEG_PAD,), jnp.float32),
    ],
)


def _layer_body(y_hbm, row_hbm, col_hbm, ew_hbm, out_hbm,
                row_v, col_v, ew_v, msg_v, gsem, acc):
    c = lax.axis_index("c")
    s = lax.axis_index("s")

    pltpu.sync_copy(y_hbm.at[pl.ds(c * N + s * RPT, RPT)],
                    acc.at[pl.ds(s * RPT, RPT)])

    @pl.when(s == 0)
    def _():
        pltpu.sync_copy(y_hbm.at[pl.ds(c * N + NS * RPT, TAIL)],
                        acc.at[pl.ds(NS * RPT, TAIL)])

    pltpu.sync_copy(row_hbm.at[s], row_v)
    pltpu.sync_copy(col_hbm.at[s], col_v)
    pltpu.sync_copy(ew_hbm.at[s], ew_v)

    offv = jnp.full((L,), c * N, jnp.int32)

    def addoff(i, carry):
        for u in range(CK // L):
            sl = (i, pl.ds(u * L, L))
            row_v[sl] = row_v[sl] + offv
        return carry

    lax.fori_loop(0, NCHUNK, addoff, 0)
    plsc.subcore_barrier()

    HC = CK // 2

    def gissue(i, hb):
        pltpu.async_copy(y_hbm.at[row_v.at[i, pl.ds(hb * HC, HC)]],
                         msg_v.at[pl.ds(hb * HC, HC)], gsem)

    def gwait(i, hb):
        pltpu.make_async_copy(y_hbm.at[row_v.at[i, pl.ds(hb * HC, HC)]],
                              msg_v.at[pl.ds(hb * HC, HC)], gsem).wait()

    def half(i, hb):
        def scale(g, carry2):
            ew16 = ew_v[i, pl.ds(hb * HC + g * L, L)]
            for lane in range(L):
                wv = jnp.full((L,), ew16[lane], jnp.float32)
                j = hb * HC + g * L + lane
                for u in range(DH // L):
                    sl = (j, pl.ds(u * L, L))
                    msg_v[sl] = msg_v[sl] * wv
            return carry2

        lax.fori_loop(0, HC // L, scale, 0)
        pltpu.sync_copy(msg_v.at[pl.ds(hb * HC, HC)],
                        acc.at[col_v.at[i, pl.ds(hb * HC, HC)]], add=True)

    gissue(0, 0)

    def chunk(i, carry):
        gwait(i, 0)
        gissue(i, 1)
        half(i, 0)
        gwait(i, 1)

        @pl.when(i + 1 < NCHUNK)
        def _():
            gissue(i + 1, 0)

        half(i, 1)
        return carry

    lax.fori_loop(0, NCHUNK, chunk, 0)
    plsc.subcore_barrier()
    pltpu.sync_copy(acc.at[pl.ds(s * RPT, RPT)],
                    out_hbm.at[pl.ds(c * N + s * RPT, RPT)])

    @pl.when(s == 0)
    def _():
        pltpu.sync_copy(acc.at[pl.ds(NS * RPT, TAIL)],
                        out_hbm.at[pl.ds(c * N + NS * RPT, TAIL)])


_layer_kernel = pl.kernel(
    _layer_body,
    out_type=jax.ShapeDtypeStruct((NC * N, DH), jnp.float32),
    mesh=_mesh,
    scratch_types=[
        pltpu.VMEM((NCHUNK, CK), jnp.int32),
        pltpu.VMEM((NCHUNK, CK), jnp.int32),
        pltpu.VMEM((NCHUNK, CK), jnp.float32),
        pltpu.VMEM((CK, DH), jnp.float32),
        pltpu.SemaphoreType.DMA,
        pltpu.VMEM_SHARED((N, DH), jnp.float32),
    ],
)


BR = 1000


def _first_body(x_ref, w_ref, dinv_ref, out_ref):
    xw = jnp.dot(x_ref[...], w_ref[...], preferred_element_type=jnp.float32)
    y = xw * dinv_ref[...]
    out_ref[0] = y[:, :DH]
    out_ref[1] = y[:, DH:]


_first_kernel = pl.pallas_call(
    _first_body,
    grid=(N // BR,),
    in_specs=[
        pl.BlockSpec((BR, D), lambda i: (i, 0)),
        pl.BlockSpec((D, D), lambda i: (0, 0)),
        pl.BlockSpec((BR, 1), lambda i: (i, 0)),
    ],
    out_specs=pl.BlockSpec((NC, BR, DH), lambda i: (0, i, 0)),
    out_shape=jax.ShapeDtypeStruct((NC, N, DH), jnp.float32),
)


def _mid_body(a_ref, dinv_ref, b_ref, w_ref, out_ref):
    d = dinv_ref[...]
    h0 = jnp.maximum(a_ref[0] * d + b_ref[:, :DH], 0.0)
    h1 = jnp.maximum(a_ref[1] * d + b_ref[:, DH:], 0.0)
    y = (jnp.dot(h0, w_ref[:DH, :], preferred_element_type=jnp.float32)
         + jnp.dot(h1, w_ref[DH:, :], preferred_element_type=jnp.float32)) * d
    out_ref[0] = y[:, :DH]
    out_ref[1] = y[:, DH:]


_mid_kernel = pl.pallas_call(
    _mid_body,
    grid=(N // BR,),
    in_specs=[
        pl.BlockSpec((NC, BR, DH), lambda i: (0, i, 0)),
        pl.BlockSpec((BR, 1), lambda i: (i, 0)),
        pl.BlockSpec((1, D), lambda i: (0, 0)),
        pl.BlockSpec((D, D), lambda i: (0, 0)),
    ],
    out_specs=pl.BlockSpec((NC, BR, DH), lambda i: (0, i, 0)),
    out_shape=jax.ShapeDtypeStruct((NC, N, DH), jnp.float32),
)


def _last_body(a_ref, dinv_ref, b_ref, out_ref):
    d = dinv_ref[...]
    out_ref[:, :DH] = jax.nn.sigmoid(a_ref[0] * d + b_ref[:, :DH])
    out_ref[:, DH:] = jax.nn.sigmoid(a_ref[1] * d + b_ref[:, DH:])


_last_kernel = pl.pallas_call(
    _last_body,
    grid=(N // BR,),
    in_specs=[
        pl.BlockSpec((NC, BR, DH), lambda i: (0, i, 0)),
        pl.BlockSpec((BR, 1), lambda i: (i, 0)),
        pl.BlockSpec((1, D), lambda i: (0, 0)),
    ],
    out_specs=pl.BlockSpec((BR, D), lambda i: (i, 0)),
    out_shape=jax.ShapeDtypeStruct((N, D), jnp.float32),
)


def kernel(x, edge_index, edge_attr, W1, b1, W2, b2, W3, b3):
    row = edge_index[0].astype(jnp.int32)
    col = edge_index[1].astype(jnp.int32)
    ew = edge_attr.astype(jnp.float32)
    pad = EPAD - E
    rowp = jnp.concatenate([row, jnp.zeros((pad,), jnp.int32)]
                           ).reshape(NS, NCHUNK, CK)
    colp = jnp.concatenate([col, jnp.zeros((pad,), jnp.int32)]
                           ).reshape(NS, NCHUNK, CK)
    ewp = jnp.concatenate([ew, jnp.zeros((pad,), jnp.float32)]
                          ).reshape(NS, NCHUNK, CK)

    degp = _deg_kernel(colp, ewp).reshape(NC, DEG_PAD)
    deg = degp[0, :N] + degp[1, :N] + 1.0
    dinv = jnp.where(deg > 0, lax.rsqrt(jnp.maximum(deg, 1e-30)),
                     0.0).reshape(N, 1)

    y1 = _first_kernel(x, W1, dinv).reshape(NC * N, DH)
    a1 = _layer_kernel(y1, rowp, colp, ewp).reshape(NC, N, DH)
    y2 = _mid_kernel(a1, dinv, b1.reshape(1, D), W2).reshape(NC * N, DH)
    a2 = _layer_kernel(y2, rowp, colp, ewp).reshape(NC, N, DH)
    y3 = _mid_kernel(a2, dinv, b2.reshape(1, D), W3).reshape(NC * N, DH)
    a3 = _layer_kernel(y3, rowp, colp, ewp).reshape(NC, N, DH)
    return _last_kernel(a3, dinv, b3.reshape(1, D))
